# baseline (speedup 1.0000x reference)
import jax
import jax.numpy as jnp
from jax.experimental import pallas as pl
from jax.experimental.pallas import tpu as pltpu

N, C, H, WD = 2, 192, 224, 224
CO = 384
TILE_H = 8
WP = 256
MT = TILE_H * WP
N_TILES = H // TILE_H


def _conv_body(x_ref, w_ref, b_ref, o_ref):
    xb = x_ref[0, :, :TILE_H + 2, :].astype(jnp.bfloat16)
    xt = jnp.transpose(xb, (1, 0, 2))
    z32 = jnp.zeros((C, WP - WD), jnp.bfloat16)
    rows = [xt[j] for j in range(TILE_H + 2)]
    pieces = []
    for r in rows:
        pieces.append(r)
        pieces.append(z32)
    cat0 = jnp.concatenate(pieces, axis=1)
    zc = jnp.zeros((C, 1), jnp.bfloat16)
    cats = (
        jnp.concatenate([zc, cat0[:, :-1]], axis=1),
        cat0,
        jnp.concatenate([cat0[:, 1:], zc], axis=1),
    )
    acc = jnp.broadcast_to(b_ref[...], (CO, MT)).astype(jnp.float32)
    for kh in range(3):
        for kw in range(3):
            rhs = cats[kw][:, kh * WP:kh * WP + MT]
            acc = acc + jnp.dot(w_ref[kh * 3 + kw], rhs,
                                preferred_element_type=jnp.float32)
    out = jnp.stack([acc[:, r * WP:r * WP + WD] for r in range(TILE_H)], axis=1)
    o_ref[0] = out


@jax.jit
def kernel(x, W, b):
    wt = jnp.transpose(W, (2, 3, 0, 1)).reshape(9, CO, C).astype(jnp.bfloat16)
    b2 = b.reshape(CO, 1)
    xp = jnp.pad(x, ((0, 0), (0, 0), (1, 7), (0, 0)))
    return pl.pallas_call(
        _conv_body,
        grid=(N, N_TILES),
        in_specs=[
            pl.BlockSpec(
                (pl.Element(1), pl.Element(C),
                 pl.Element(16), pl.Element(WD)),
                lambda n, i: (n, 0, i * TILE_H, 0),
            ),
            pl.BlockSpec((9, CO, C), lambda n, i: (0, 0, 0)),
            pl.BlockSpec((CO, 1), lambda n, i: (0, 0)),
        ],
        out_specs=pl.BlockSpec((1, CO, TILE_H, WD), lambda n, i: (n, 0, i, 0)),
        out_shape=jax.ShapeDtypeStruct((N, CO, H, WD), jnp.float32),
        compiler_params=pltpu.CompilerParams(
            dimension_semantics=("parallel", "parallel"),
        ),
    )(xp, wt, b2)


# speedup vs baseline: 1.1851x; 1.1851x over previous
import jax
import jax.numpy as jnp
from jax.experimental import pallas as pl
from jax.experimental.pallas import tpu as pltpu

N, C, H, WD = 2, 192, 224, 224
CO = 384
TILE_H = 8
WP = 256
MT = TILE_H * WP
N_TILES = H // TILE_H


def _conv_body(x_ref, w_ref, b_ref, o_ref):
    xb = x_ref[0, :, :TILE_H + 2, :].astype(jnp.bfloat16)
    xt = jnp.transpose(xb, (1, 0, 2))
    z32 = jnp.zeros((C, WP - WD), jnp.bfloat16)
    rows = [xt[j] for j in range(TILE_H + 2)]
    pieces = []
    for r in rows:
        pieces.append(r)
        pieces.append(z32)
    cat0 = jnp.concatenate(pieces, axis=1)
    zc = jnp.zeros((C, 1), jnp.bfloat16)
    cats = (
        jnp.concatenate([zc, cat0[:, :-1]], axis=1),
        cat0,
        jnp.concatenate([cat0[:, 1:], zc], axis=1),
    )
    slabs = [cats[kw][:, kh * WP:kh * WP + MT]
             for kh in range(3) for kw in range(3)]
    rhs = jnp.concatenate(slabs, axis=0)
    acc = jnp.dot(w_ref[...], rhs, preferred_element_type=jnp.float32)
    acc = acc + jnp.broadcast_to(b_ref[...], (CO, MT)).astype(jnp.float32)
    out = jnp.stack([acc[:, r * WP:r * WP + WD] for r in range(TILE_H)], axis=1)
    o_ref[0] = out


@jax.jit
def kernel(x, W, b):
    wt = jnp.transpose(W, (0, 2, 3, 1)).reshape(CO, 9 * C).astype(jnp.bfloat16)
    b2 = b.reshape(CO, 1)
    xp = jnp.pad(x, ((0, 0), (0, 0), (1, 7), (0, 0)))
    return pl.pallas_call(
        _conv_body,
        grid=(N, N_TILES),
        in_specs=[
            pl.BlockSpec(
                (pl.Element(1), pl.Element(C),
                 pl.Element(16), pl.Element(WD)),
                lambda n, i: (n, 0, i * TILE_H, 0),
            ),
            pl.BlockSpec((CO, 9 * C), lambda n, i: (0, 0)),
            pl.BlockSpec((CO, 1), lambda n, i: (0, 0)),
        ],
        out_specs=pl.BlockSpec((1, CO, TILE_H, WD), lambda n, i: (n, 0, i, 0)),
        out_shape=jax.ShapeDtypeStruct((N, CO, H, WD), jnp.float32),
        compiler_params=pltpu.CompilerParams(
            dimension_semantics=("parallel", "parallel"),
        ),
    )(xp, wt, b2)


# speedup vs baseline: 1.2825x; 1.0822x over previous
import jax
import jax.numpy as jnp
from jax.experimental import pallas as pl
from jax.experimental.pallas import tpu as pltpu

N, C, H, WD = 2, 192, 224, 224
CO = 384
TILE_H = 8
WP = 256
MT = TILE_H * WP
N_TILES = H // TILE_H
BLK_H = 24
MAX_OFF = H - BLK_H


def _conv_body(x_ref, w_ref, b_ref, o_ref):
    i = pl.program_id(1)
    xb = x_ref[0].astype(jnp.bfloat16)
    xt = jnp.transpose(xb, (1, 0, 2))
    z32 = jnp.zeros((C, WP - WD), jnp.bfloat16)
    z256 = jnp.zeros((C, WP), jnp.bfloat16)
    pieces = [z256]
    for j in range(BLK_H):
        pieces.append(xt[j])
        pieces.append(z32)
    pieces.append(z256)
    catall = jnp.concatenate(pieces, axis=1)
    cat_top = catall[:, 0:10 * WP]
    cat_mid = catall[:, 8 * WP:18 * WP]
    cat_bot = catall[:, 16 * WP:26 * WP]
    cat = jnp.where(i == 0, cat_top, cat_mid)
    cat = jnp.where(i == N_TILES - 1, cat_bot, cat)
    zc = jnp.zeros((C, 1), jnp.bfloat16)
    cats = (
        jnp.concatenate([zc, cat[:, :-1]], axis=1),
        cat,
        jnp.concatenate([cat[:, 1:], zc], axis=1),
    )
    slabs = [cats[kw][:, kh * WP:kh * WP + MT]
             for kh in range(3) for kw in range(3)]
    rhs = jnp.concatenate(slabs, axis=0)
    acc = jnp.dot(w_ref[...], rhs, preferred_element_type=jnp.float32)
    acc = acc + jnp.broadcast_to(b_ref[...], (CO, MT)).astype(jnp.float32)
    out = jnp.stack([acc[:, r * WP:r * WP + WD] for r in range(TILE_H)], axis=1)
    o_ref[0] = out


@jax.jit
def kernel(x, W, b):
    wt = jnp.transpose(W, (0, 2, 3, 1)).reshape(CO, 9 * C).astype(jnp.bfloat16)
    b2 = b.reshape(CO, 1)
    return pl.pallas_call(
        _conv_body,
        grid=(N, N_TILES),
        in_specs=[
            pl.BlockSpec(
                (pl.Element(1), pl.Element(C),
                 pl.Element(BLK_H), pl.Element(WD)),
                lambda n, i: (n, 0, jnp.clip(i - 1, 0, MAX_OFF // TILE_H) * TILE_H, 0),
            ),
            pl.BlockSpec((CO, 9 * C), lambda n, i: (0, 0)),
            pl.BlockSpec((CO, 1), lambda n, i: (0, 0)),
        ],
        out_specs=pl.BlockSpec((1, CO, TILE_H, WD), lambda n, i: (n, 0, i, 0)),
        out_shape=jax.ShapeDtypeStruct((N, CO, H, WD), jnp.float32),
        compiler_params=pltpu.CompilerParams(
            dimension_semantics=("parallel", "parallel"),
        ),
    )(x, wt, b2)


# speedup vs baseline: 1.3314x; 1.0381x over previous
import jax
import jax.numpy as jnp
from jax.experimental import pallas as pl
from jax.experimental.pallas import tpu as pltpu

N, C, H, WD = 2, 192, 224, 224
CO = 384
TILE_H = 8
WP = 256
MT = TILE_H * WP
N_TILES = H // TILE_H
BLK_H = 16


def _conv_body(x_ref, w_ref, b_ref, o_ref, carry_ref):
    i = pl.program_id(1)
    xb = x_ref[0, :, :9, :].astype(jnp.bfloat16)
    xt = jnp.transpose(xb, (1, 0, 2))
    rows = [jnp.where(i > 0, carry_ref[...], 0)]
    rows += [xt[j] for j in range(9)]
    rows[9] = jnp.where(i < N_TILES - 1, rows[9], 0)
    z32 = jnp.zeros((C, WP - WD), jnp.bfloat16)
    pieces = []
    for r in rows:
        pieces.append(r)
        pieces.append(z32)
    cat = jnp.concatenate(pieces, axis=1)
    carry_ref[...] = xt[7]
    zc = jnp.zeros((C, 1), jnp.bfloat16)
    cats = (
        jnp.concatenate([zc, cat[:, :-1]], axis=1),
        cat,
        jnp.concatenate([cat[:, 1:], zc], axis=1),
    )
    slabs = [cats[kw][:, kh * WP:kh * WP + MT]
             for kh in range(3) for kw in range(3)]
    rhs = jnp.concatenate(slabs, axis=0)
    acc = jnp.dot(w_ref[...], rhs, preferred_element_type=jnp.float32)
    acc = acc + jnp.broadcast_to(b_ref[...], (CO, MT)).astype(jnp.float32)
    out = jnp.stack([acc[:, r * WP:r * WP + WD] for r in range(TILE_H)], axis=1)
    o_ref[0] = out


@jax.jit
def kernel(x, W, b):
    wt = jnp.transpose(W, (0, 2, 3, 1)).reshape(CO, 9 * C).astype(jnp.bfloat16)
    b2 = b.reshape(CO, 1)
    return pl.pallas_call(
        _conv_body,
        grid=(N, N_TILES),
        in_specs=[
            pl.BlockSpec(
                (pl.Element(1), pl.Element(C),
                 pl.Element(BLK_H, padding=(0, 8)), pl.Element(WD)),
                lambda n, i: (n, 0, i * TILE_H, 0),
            ),
            pl.BlockSpec((CO, 9 * C), lambda n, i: (0, 0)),
            pl.BlockSpec((CO, 1), lambda n, i: (0, 0)),
        ],
        out_specs=pl.BlockSpec((1, CO, TILE_H, WD), lambda n, i: (n, 0, i, 0)),
        out_shape=jax.ShapeDtypeStruct((N, CO, H, WD), jnp.float32),
        scratch_shapes=[pltpu.VMEM((C, WD), jnp.bfloat16)],
        compiler_params=pltpu.CompilerParams(
            dimension_semantics=("arbitrary", "arbitrary"),
        ),
    )(x, wt, b2)
